# speedup vs baseline: 71.7631x; 71.7631x over previous
import functools

import jax
import jax.numpy as jnp
from jax.experimental import pallas as pl
from jax.experimental.pallas import tpu as pltpu


def _body(k_ref, tau_ref, x_ref, w_ref, b_ref, o_ref):
    k = k_ref[0]
    tau = tau_ref[0]
    xb = x_ref[...]
    bits = jax.lax.bitcast_convert_type(xb, jnp.int32)
    vi = jnp.where(bits >= 0, bits, bits ^ jnp.int32(0x7FFFFFFF))

    s1 = jnp.sum(xb, axis=1, keepdims=True)

    one = jnp.int32(1)

    def step(i, t):
        cand = t + jnp.left_shift(one, 31 - i)
        cnt = jnp.sum((vi >= cand).astype(jnp.int32), axis=1, keepdims=True)
        return jnp.where(cnt >= k, cand, t)

    t0 = jnp.full((xb.shape[0], 1), jnp.iinfo(jnp.int32).min, jnp.int32)
    tsel = jax.lax.fori_loop(0, 32, step, t0)

    tbits = jnp.where(tsel >= 0, tsel, tsel ^ jnp.int32(0x7FFFFFFF))
    tval = jax.lax.bitcast_convert_type(tbits, jnp.float32)

    gt = vi > tsel
    cnt_gt = jnp.sum(gt.astype(jnp.int32), axis=1, keepdims=True)
    s2 = (jnp.sum(jnp.where(gt, xb, 0.0), axis=1, keepdims=True)
          + tval * (k - cnt_gt).astype(jnp.float32))

    scale = jnp.exp(s1 / s2) / tau

    y = jnp.dot(xb, w_ref[...], preferred_element_type=jnp.float32)
    o_ref[...] = y * scale + b_ref[...]


def kernel(x, W, b, percentile, tau):
    B, D = x.shape
    C = W.shape[1]
    BR = 256
    k_arr = (D - jnp.round(D * percentile / 100.0)).astype(jnp.int32).reshape(1)
    tau_arr = jnp.asarray(tau, jnp.float32).reshape(1)
    grid = (B // BR,)
    return pl.pallas_call(
        _body,
        grid=grid,
        in_specs=[
            pl.BlockSpec(memory_space=pltpu.SMEM),
            pl.BlockSpec(memory_space=pltpu.SMEM),
            pl.BlockSpec((BR, D), lambda i: (i, 0)),
            pl.BlockSpec((D, C), lambda i: (0, 0)),
            pl.BlockSpec((1, C), lambda i: (0, 0)),
        ],
        out_specs=pl.BlockSpec((BR, C), lambda i: (i, 0)),
        out_shape=jax.ShapeDtypeStruct((B, C), jnp.float32),
    )(k_arr, tau_arr, x, W, b.reshape(1, C))


# speedup vs baseline: 72.1417x; 1.0053x over previous
import functools

import jax
import jax.numpy as jnp
from jax.experimental import pallas as pl
from jax.experimental.pallas import tpu as pltpu


def _body(k_ref, tau_ref, x_ref, w_ref, b_ref, o_ref):
    k = k_ref[0]
    tau = tau_ref[0]
    xb = x_ref[...]
    bits = jax.lax.bitcast_convert_type(xb, jnp.int32)
    vi = jnp.where(bits >= 0, bits, bits ^ jnp.int32(0x7FFFFFFF))

    s1 = jnp.sum(xb, axis=1, keepdims=True)

    one = jnp.int32(1)

    def step(i, t):
        cand = t + jnp.left_shift(one, 31 - i)
        cnt = jnp.sum((vi >= cand).astype(jnp.int32), axis=1, keepdims=True)
        return jnp.where(cnt >= k, cand, t)

    t0 = jnp.full((xb.shape[0], 1), jnp.iinfo(jnp.int32).min, jnp.int32)
    tsel = jax.lax.fori_loop(0, 32, step, t0)

    tbits = jnp.where(tsel >= 0, tsel, tsel ^ jnp.int32(0x7FFFFFFF))
    tval = jax.lax.bitcast_convert_type(tbits, jnp.float32)

    gt = vi > tsel
    cnt_gt = jnp.sum(gt.astype(jnp.int32), axis=1, keepdims=True)
    s2 = (jnp.sum(jnp.where(gt, xb, 0.0), axis=1, keepdims=True)
          + tval * (k - cnt_gt).astype(jnp.float32))

    scale = jnp.exp(s1 / s2) / tau

    y = jnp.dot(xb.astype(jnp.bfloat16), w_ref[...],
                preferred_element_type=jnp.float32)
    o_ref[...] = y * scale + b_ref[...]


def kernel(x, W, b, percentile, tau):
    B, D = x.shape
    C = W.shape[1]
    BR = 256
    k_arr = (D - jnp.round(D * percentile / 100.0)).astype(jnp.int32).reshape(1)
    tau_arr = jnp.asarray(tau, jnp.float32).reshape(1)
    grid = (B // BR,)
    return pl.pallas_call(
        _body,
        grid=grid,
        in_specs=[
            pl.BlockSpec(memory_space=pltpu.SMEM),
            pl.BlockSpec(memory_space=pltpu.SMEM),
            pl.BlockSpec((BR, D), lambda i: (i, 0)),
            pl.BlockSpec((D, C), lambda i: (0, 0)),
            pl.BlockSpec((1, C), lambda i: (0, 0)),
        ],
        out_specs=pl.BlockSpec((BR, C), lambda i: (i, 0)),
        out_shape=jax.ShapeDtypeStruct((B, C), jnp.float32),
    )(k_arr, tau_arr, x, W.astype(jnp.bfloat16), b.reshape(1, C))


# speedup vs baseline: 143.9883x; 1.9959x over previous
import functools

import jax
import jax.numpy as jnp
from jax.experimental import pallas as pl
from jax.experimental.pallas import tpu as pltpu


def _body(k_ref, tau_ref, x_ref, w_ref, b_ref, o_ref):
    ITERS = 16
    k = k_ref[0]
    tau = tau_ref[0]
    xb = x_ref[...]
    bits = jax.lax.bitcast_convert_type(xb, jnp.int32)
    vi = jnp.where(bits >= 0, bits, bits ^ jnp.int32(0x7FFFFFFF))

    s1 = jnp.sum(xb, axis=1, keepdims=True)

    tsel = jnp.full((xb.shape[0], 1), jnp.iinfo(jnp.int32).min, jnp.int32)
    for i in range(ITERS):
        step = 1 << (31 - i)
        cand = tsel + jnp.int32(step - (1 << 32) if step >= (1 << 31) else step)
        cnt = jnp.sum((vi >= cand).astype(jnp.int32), axis=1, keepdims=True)
        tsel = jnp.where(cnt >= k, cand, tsel)

    hi_mask = jnp.int32(-(1 << (32 - ITERS)))
    gt = (vi & hi_mask) > tsel
    mid = tsel | jnp.int32((1 << (31 - ITERS)) if ITERS < 32 else 0)
    tbits = jnp.where(mid >= 0, mid, mid ^ jnp.int32(0x7FFFFFFF))
    tval = jax.lax.bitcast_convert_type(tbits, jnp.float32)

    cnt_gt = jnp.sum(gt.astype(jnp.int32), axis=1, keepdims=True)
    s2 = (jnp.sum(jnp.where(gt, xb, 0.0), axis=1, keepdims=True)
          + tval * (k - cnt_gt).astype(jnp.float32))

    scale = jnp.exp(s1 / s2) / tau

    y = jnp.dot(xb.astype(jnp.bfloat16), w_ref[...],
                preferred_element_type=jnp.float32)
    o_ref[...] = y * scale + b_ref[...]


def kernel(x, W, b, percentile, tau):
    B, D = x.shape
    C = W.shape[1]
    BR = 256
    k_arr = (D - jnp.round(D * percentile / 100.0)).astype(jnp.int32).reshape(1)
    tau_arr = jnp.asarray(tau, jnp.float32).reshape(1)
    grid = (B // BR,)
    return pl.pallas_call(
        _body,
        grid=grid,
        in_specs=[
            pl.BlockSpec(memory_space=pltpu.SMEM),
            pl.BlockSpec(memory_space=pltpu.SMEM),
            pl.BlockSpec((BR, D), lambda i: (i, 0)),
            pl.BlockSpec((D, C), lambda i: (0, 0)),
            pl.BlockSpec((1, C), lambda i: (0, 0)),
        ],
        out_specs=pl.BlockSpec((BR, C), lambda i: (i, 0)),
        out_shape=jax.ShapeDtypeStruct((B, C), jnp.float32),
    )(k_arr, tau_arr, x, W.astype(jnp.bfloat16), b.reshape(1, C))


# speedup vs baseline: 168.2651x; 1.1686x over previous
import functools

import jax
import jax.numpy as jnp
from jax.experimental import pallas as pl
from jax.experimental.pallas import tpu as pltpu


def _body(k_ref, tau_ref, x_ref, w_ref, b_ref, o_ref):
    ITERS = 12
    k = k_ref[0]
    tau = tau_ref[0]
    xb = x_ref[...]
    bits = jax.lax.bitcast_convert_type(xb, jnp.int32)
    vi = jnp.where(bits >= 0, bits, bits ^ jnp.int32(0x7FFFFFFF))

    s1 = jnp.sum(xb, axis=1, keepdims=True)

    tsel = jnp.full((xb.shape[0], 1), jnp.iinfo(jnp.int32).min, jnp.int32)
    for i in range(ITERS):
        step = 1 << (31 - i)
        cand = tsel + jnp.int32(step - (1 << 32) if step >= (1 << 31) else step)
        cnt = jnp.sum((vi >= cand).astype(jnp.int32), axis=1, keepdims=True)
        tsel = jnp.where(cnt >= k, cand, tsel)

    hi_mask = jnp.int32(-(1 << (32 - ITERS)))
    gt = (vi & hi_mask) > tsel
    mid = tsel | jnp.int32((1 << (31 - ITERS)) if ITERS < 32 else 0)
    tbits = jnp.where(mid >= 0, mid, mid ^ jnp.int32(0x7FFFFFFF))
    tval = jax.lax.bitcast_convert_type(tbits, jnp.float32)

    cnt_gt = jnp.sum(gt.astype(jnp.int32), axis=1, keepdims=True)
    s2 = (jnp.sum(jnp.where(gt, xb, 0.0), axis=1, keepdims=True)
          + tval * (k - cnt_gt).astype(jnp.float32))

    scale = jnp.exp(s1 / s2) / tau

    y = jnp.dot(xb.astype(jnp.bfloat16), w_ref[...],
                preferred_element_type=jnp.float32)
    o_ref[...] = y * scale + b_ref[...]


def kernel(x, W, b, percentile, tau):
    B, D = x.shape
    C = W.shape[1]
    BR = 256
    k_arr = (D - jnp.round(D * percentile / 100.0)).astype(jnp.int32).reshape(1)
    tau_arr = jnp.asarray(tau, jnp.float32).reshape(1)
    grid = (B // BR,)
    return pl.pallas_call(
        _body,
        grid=grid,
        in_specs=[
            pl.BlockSpec(memory_space=pltpu.SMEM),
            pl.BlockSpec(memory_space=pltpu.SMEM),
            pl.BlockSpec((BR, D), lambda i: (i, 0)),
            pl.BlockSpec((D, C), lambda i: (0, 0)),
            pl.BlockSpec((1, C), lambda i: (0, 0)),
        ],
        out_specs=pl.BlockSpec((BR, C), lambda i: (i, 0)),
        out_shape=jax.ShapeDtypeStruct((B, C), jnp.float32),
    )(k_arr, tau_arr, x, W.astype(jnp.bfloat16), b.reshape(1, C))


# speedup vs baseline: 189.8070x; 1.1280x over previous
import functools

import jax
import jax.numpy as jnp
from jax.experimental import pallas as pl
from jax.experimental.pallas import tpu as pltpu


def _body(k_ref, tau_ref, x_ref, w_ref, b_ref, o_ref):
    ITERS = 12
    BR = x_ref.shape[0]
    D = x_ref.shape[1]
    k = k_ref[0]
    tau = tau_ref[0]
    xb = x_ref[...]
    xb16 = xb.astype(jnp.bfloat16)

    one_bf = jnp.bfloat16(1)
    zero_bf = jnp.bfloat16(0)
    ones_mx = jnp.ones((128, 128), jnp.bfloat16)
    kf = k.astype(jnp.float32)

    def lane_partials(a):
        parts = [a[:, j * 128:(j + 1) * 128] for j in range(D // 128)]
        while len(parts) > 1:
            parts = [parts[j] + parts[j + 1] for j in range(0, len(parts), 2)]
        return parts[0]

    def decode16(enc32):
        e = enc32.astype(jnp.int16)
        tb = jnp.where(e >= 0, e, e ^ jnp.int16(0x7FFF))
        return jax.lax.bitcast_convert_type(tb, jnp.bfloat16).astype(jnp.float32)

    def row_scale(xh, xh16):
        R = xh.shape[0]
        b16 = jax.lax.bitcast_convert_type(xh16, jnp.int16)
        vi = jnp.where(b16 >= 0, b16, b16 ^ jnp.int16(0x7FFF))
        s1 = jnp.sum(xh, axis=1, keepdims=True)
        tsel = jnp.full((R, 1), -(1 << 15), jnp.int32)
        for i in range(ITERS):
            cand = tsel + jnp.int32(1 << (15 - i))
            m = jnp.where(vi >= cand.astype(jnp.int16), one_bf, zero_bf)
            p = lane_partials(m)
            c = jnp.dot(p, ones_mx, preferred_element_type=jnp.float32)
            tsel = jnp.where(c[:, :1] >= kf, cand, tsel)
        tval = decode16(tsel | jnp.int32((1 << (15 - ITERS)) if ITERS < 16 else 0))
        thr_enc = (tsel + jnp.int32(1 << (16 - ITERS))).astype(jnp.int16)
        mgt = jnp.where(vi >= thr_enc, one_bf, zero_bf)
        cg = jnp.dot(lane_partials(mgt), ones_mx,
                     preferred_element_type=jnp.float32)[:, :1]
        sx = jnp.dot(lane_partials(mgt * xh16), ones_mx,
                     preferred_element_type=jnp.float32)[:, :1]
        s2 = sx + tval * (kf - cg)
        return jnp.exp(s1 / s2) / tau

    scale = row_scale(xb, xb16)

    y = jnp.dot(xb16, w_ref[...], preferred_element_type=jnp.float32)
    o_ref[...] = y * scale + b_ref[...]


def kernel(x, W, b, percentile, tau):
    B, D = x.shape
    C = W.shape[1]
    BR = 256
    k_arr = (D - jnp.round(D * percentile / 100.0)).astype(jnp.int32).reshape(1)
    tau_arr = jnp.asarray(tau, jnp.float32).reshape(1)
    grid = (B // BR,)
    return pl.pallas_call(
        _body,
        grid=grid,
        in_specs=[
            pl.BlockSpec(memory_space=pltpu.SMEM),
            pl.BlockSpec(memory_space=pltpu.SMEM),
            pl.BlockSpec((BR, D), lambda i: (i, 0)),
            pl.BlockSpec((D, C), lambda i: (0, 0)),
            pl.BlockSpec((1, C), lambda i: (0, 0)),
        ],
        out_specs=pl.BlockSpec((BR, C), lambda i: (i, 0)),
        out_shape=jax.ShapeDtypeStruct((B, C), jnp.float32),
    )(k_arr, tau_arr, x, W.astype(jnp.bfloat16), b.reshape(1, C))


# speedup vs baseline: 223.8158x; 1.1792x over previous
import functools

import jax
import jax.numpy as jnp
from jax.experimental import pallas as pl
from jax.experimental.pallas import tpu as pltpu


def _body(k_ref, tau_ref, x_ref, w_ref, b_ref, o_ref):
    ITERS = 12
    BR = x_ref.shape[0]
    D = x_ref.shape[1]
    k = k_ref[0]
    tau = tau_ref[0]
    xb = x_ref[...]
    xb16 = xb.astype(jnp.bfloat16)

    one_bf = jnp.bfloat16(1)
    zero_bf = jnp.bfloat16(0)
    ones_mx = jnp.ones((128, 128), jnp.bfloat16)
    kf = k.astype(jnp.float32)

    def lane_partials(a):
        parts = [a[:, j * 128:(j + 1) * 128] for j in range(D // 128)]
        while len(parts) > 1:
            parts = [parts[j] + parts[j + 1] for j in range(0, len(parts), 2)]
        return parts[0]

    def decode16(enc32):
        e = enc32.astype(jnp.int16)
        tb = jnp.where(e >= 0, e, e ^ jnp.int16(0x7FFF))
        return jax.lax.bitcast_convert_type(tb, jnp.bfloat16).astype(jnp.float32)

    def row_scale(xh, xh16):
        R = xh.shape[0]
        b16 = jax.lax.bitcast_convert_type(xh16, jnp.int16)
        vi = jnp.where(b16 >= 0, b16, b16 ^ jnp.int16(0x7FFF))
        s1 = jnp.sum(xh, axis=1, keepdims=True)
        tsel = jnp.full((R, 1), -(1 << 15), jnp.int32)
        for i in range(ITERS):
            cand = tsel + jnp.int32(1 << (15 - i))
            m = jnp.where(vi >= cand.astype(jnp.int16), one_bf, zero_bf)
            p = lane_partials(m)
            c = jnp.dot(p, ones_mx, preferred_element_type=jnp.float32)
            tsel = jnp.where(c[:, :1] >= kf, cand, tsel)
        tval = decode16(tsel | jnp.int32((1 << (15 - ITERS)) if ITERS < 16 else 0))
        thr_enc = (tsel + jnp.int32(1 << (16 - ITERS))).astype(jnp.int16)
        mgt = jnp.where(vi >= thr_enc, one_bf, zero_bf)
        cg = jnp.dot(lane_partials(mgt), ones_mx,
                     preferred_element_type=jnp.float32)[:, :1]
        sx = jnp.dot(lane_partials(mgt * xh16), ones_mx,
                     preferred_element_type=jnp.float32)[:, :1]
        s2 = sx + tval * (kf - cg)
        return jnp.exp(s1 / s2) / tau

    scale = row_scale(xb, xb16)

    y = jnp.dot(xb16, w_ref[...], preferred_element_type=jnp.float32)
    o_ref[...] = y * scale + b_ref[...]


def kernel(x, W, b, percentile, tau):
    B, D = x.shape
    C = W.shape[1]
    BR = 512
    k_arr = (D - jnp.round(D * percentile / 100.0)).astype(jnp.int32).reshape(1)
    tau_arr = jnp.asarray(tau, jnp.float32).reshape(1)
    grid = (B // BR,)
    return pl.pallas_call(
        _body,
        grid=grid,
        in_specs=[
            pl.BlockSpec(memory_space=pltpu.SMEM),
            pl.BlockSpec(memory_space=pltpu.SMEM),
            pl.BlockSpec((BR, D), lambda i: (i, 0)),
            pl.BlockSpec((D, C), lambda i: (0, 0)),
            pl.BlockSpec((1, C), lambda i: (0, 0)),
        ],
        out_specs=pl.BlockSpec((BR, C), lambda i: (i, 0)),
        out_shape=jax.ShapeDtypeStruct((B, C), jnp.float32),
    )(k_arr, tau_arr, x, W.astype(jnp.bfloat16), b.reshape(1, C))


# speedup vs baseline: 241.1781x; 1.0776x over previous
import functools

import jax
import jax.numpy as jnp
from jax.experimental import pallas as pl
from jax.experimental.pallas import tpu as pltpu


def _body(k_ref, tau_ref, x_ref, w_ref, b_ref, o_ref):
    ITERS = 12
    BR = x_ref.shape[0]
    D = x_ref.shape[1]
    k = k_ref[0]
    tau = tau_ref[0]
    xb = x_ref[...]
    xb16 = xb.astype(jnp.bfloat16)

    one_bf = jnp.bfloat16(1)
    zero_bf = jnp.bfloat16(0)
    ones_mx = jnp.ones((128, 128), jnp.bfloat16)
    kf = k.astype(jnp.float32)

    def lane_partials(a):
        parts = [a[:, j * 128:(j + 1) * 128] for j in range(D // 128)]
        while len(parts) > 1:
            parts = [parts[j] + parts[j + 1] for j in range(0, len(parts), 2)]
        return parts[0]

    def decode16(enc32):
        e = enc32.astype(jnp.int16)
        tb = jnp.where(e >= 0, e, e ^ jnp.int16(0x7FFF))
        return jax.lax.bitcast_convert_type(tb, jnp.bfloat16).astype(jnp.float32)

    def row_scale(xh, xh16):
        R = xh.shape[0]
        b16 = jax.lax.bitcast_convert_type(xh16, jnp.int16)
        vi = jnp.where(b16 >= 0, b16, b16 ^ jnp.int16(0x7FFF))
        s1 = jnp.sum(xh, axis=1, keepdims=True)
        tsel = jnp.full((R, 1), -(1 << 15), jnp.int32)
        for i in range(ITERS):
            cand = tsel + jnp.int32(1 << (15 - i))
            m = jnp.where(vi >= cand.astype(jnp.int16), one_bf, zero_bf)
            p = lane_partials(m)
            c = jnp.dot(p, ones_mx, preferred_element_type=jnp.float32)
            tsel = jnp.where(c[:, :1] >= kf, cand, tsel)
        tval = decode16(tsel | jnp.int32((1 << (15 - ITERS)) if ITERS < 16 else 0))
        thr_enc = (tsel + jnp.int32(1 << (16 - ITERS))).astype(jnp.int16)
        mgt = jnp.where(vi >= thr_enc, one_bf, zero_bf)
        cg = jnp.dot(lane_partials(mgt), ones_mx,
                     preferred_element_type=jnp.float32)[:, :1]
        sx = jnp.dot(lane_partials(mgt * xh16), ones_mx,
                     preferred_element_type=jnp.float32)[:, :1]
        s2 = sx + tval * (kf - cg)
        return jnp.exp(s1 / s2) / tau

    scale = row_scale(xb, xb16)

    y = jnp.dot(xb16, w_ref[...], preferred_element_type=jnp.float32)
    o_ref[...] = y * scale + b_ref[...]


def kernel(x, W, b, percentile, tau):
    B, D = x.shape
    C = W.shape[1]
    BR = 1024
    k_arr = (D - jnp.round(D * percentile / 100.0)).astype(jnp.int32).reshape(1)
    tau_arr = jnp.asarray(tau, jnp.float32).reshape(1)
    grid = (B // BR,)
    return pl.pallas_call(
        _body,
        grid=grid,
        in_specs=[
            pl.BlockSpec(memory_space=pltpu.SMEM),
            pl.BlockSpec(memory_space=pltpu.SMEM),
            pl.BlockSpec((BR, D), lambda i: (i, 0)),
            pl.BlockSpec((D, C), lambda i: (0, 0)),
            pl.BlockSpec((1, C), lambda i: (0, 0)),
        ],
        out_specs=pl.BlockSpec((BR, C), lambda i: (i, 0)),
        out_shape=jax.ShapeDtypeStruct((B, C), jnp.float32),
    )(k_arr, tau_arr, x, W.astype(jnp.bfloat16), b.reshape(1, C))


# speedup vs baseline: 256.1285x; 1.0620x over previous
import functools

import jax
import jax.numpy as jnp
from jax.experimental import pallas as pl
from jax.experimental.pallas import tpu as pltpu


def _body(k_ref, tau_ref, x_ref, w_ref, b_ref, o_ref):
    ITERS = 10
    BR = x_ref.shape[0]
    D = x_ref.shape[1]
    k = k_ref[0]
    tau = tau_ref[0]
    xb = x_ref[...]
    xb16 = xb.astype(jnp.bfloat16)

    one_bf = jnp.bfloat16(1)
    zero_bf = jnp.bfloat16(0)
    ones_mx = jnp.ones((128, 128), jnp.bfloat16)
    kf = k.astype(jnp.float32)

    def lane_partials(a):
        parts = [a[:, j * 128:(j + 1) * 128] for j in range(D // 128)]
        while len(parts) > 1:
            parts = [parts[j] + parts[j + 1] for j in range(0, len(parts), 2)]
        return parts[0]

    def decode16(enc32):
        e = enc32.astype(jnp.int16)
        tb = jnp.where(e >= 0, e, e ^ jnp.int16(0x7FFF))
        return jax.lax.bitcast_convert_type(tb, jnp.bfloat16).astype(jnp.float32)

    def row_scale(xh, xh16):
        R = xh.shape[0]
        b16 = jax.lax.bitcast_convert_type(xh16, jnp.int16)
        vi = jnp.where(b16 >= 0, b16, b16 ^ jnp.int16(0x7FFF))
        s1 = jnp.sum(xh, axis=1, keepdims=True)
        tsel = jnp.full((R, 1), -(1 << 15), jnp.int32)
        for i in range(ITERS):
            cand = tsel + jnp.int32(1 << (15 - i))
            m = jnp.where(vi >= cand.astype(jnp.int16), one_bf, zero_bf)
            p = lane_partials(m)
            c = jnp.dot(p, ones_mx, preferred_element_type=jnp.float32)
            tsel = jnp.where(c[:, :1] >= kf, cand, tsel)
        tval = decode16(tsel | jnp.int32((1 << (15 - ITERS)) if ITERS < 16 else 0))
        thr_enc = (tsel + jnp.int32(1 << (16 - ITERS))).astype(jnp.int16)
        mgt = jnp.where(vi >= thr_enc, one_bf, zero_bf)
        cg = jnp.dot(lane_partials(mgt), ones_mx,
                     preferred_element_type=jnp.float32)[:, :1]
        sx = jnp.dot(lane_partials(mgt * xh16), ones_mx,
                     preferred_element_type=jnp.float32)[:, :1]
        s2 = sx + tval * (kf - cg)
        return jnp.exp(s1 / s2) / tau

    scale = row_scale(xb, xb16)

    y = jnp.dot(xb16, w_ref[...], preferred_element_type=jnp.float32)
    o_ref[...] = y * scale + b_ref[...]


def kernel(x, W, b, percentile, tau):
    B, D = x.shape
    C = W.shape[1]
    BR = 1024
    k_arr = (D - jnp.round(D * percentile / 100.0)).astype(jnp.int32).reshape(1)
    tau_arr = jnp.asarray(tau, jnp.float32).reshape(1)
    grid = (B // BR,)
    return pl.pallas_call(
        _body,
        grid=grid,
        in_specs=[
            pl.BlockSpec(memory_space=pltpu.SMEM),
            pl.BlockSpec(memory_space=pltpu.SMEM),
            pl.BlockSpec((BR, D), lambda i: (i, 0)),
            pl.BlockSpec((D, C), lambda i: (0, 0)),
            pl.BlockSpec((1, C), lambda i: (0, 0)),
        ],
        out_specs=pl.BlockSpec((BR, C), lambda i: (i, 0)),
        out_shape=jax.ShapeDtypeStruct((B, C), jnp.float32),
    )(k_arr, tau_arr, x, W.astype(jnp.bfloat16), b.reshape(1, C))
